# baseline (speedup 1.0000x reference)
import functools

import jax
import jax.numpy as jnp
from jax import lax
from jax.experimental import pallas as pl
from jax.experimental.pallas import tpu as pltpu
from jax.experimental.pallas import tpu_sc as plsc

_VOCAB = 100000
_HIDDEN = 1024
_B = 4 * 2048
_NC = 2
_NS = 16
_NW = _NC * _NS
_BPW = _B // _NW
_K = 16
_NCHUNK = _BPW // _K

_mesh = plsc.VectorSubcoreMesh(core_axis_name="c", subcore_axis_name="s")


@functools.partial(
    pl.kernel,
    mesh=_mesh,
    out_type=jax.ShapeDtypeStruct((_B, _HIDDEN), jnp.float32),
    scratch_types=[
        pltpu.VMEM((_NCHUNK, _K), jnp.int32),
        pltpu.VMEM((_K, _HIDDEN), jnp.float32),
        pltpu.VMEM((_K, _HIDDEN), jnp.float32),
        pltpu.VMEM((_K, _HIDDEN), jnp.float32),
        pltpu.SemaphoreType.DMA,
        pltpu.SemaphoreType.DMA,
        pltpu.SemaphoreType.DMA,
        pltpu.SemaphoreType.DMA,
        pltpu.SemaphoreType.DMA,
        pltpu.SemaphoreType.DMA,
    ],
)
def _gather_kernel(idx_hbm, table_hbm, out_hbm, idx_v,
                   b0, b1, b2, g0, g1, g2, o0, o1, o2):
    wid = lax.axis_index("s") * _NC + lax.axis_index("c")
    base = wid * _BPW
    bufs = (b0, b1, b2)
    gsems = (g0, g1, g2)
    osems = (o0, o1, o2)
    pltpu.sync_copy(idx_hbm.at[wid], idx_v)
    pltpu.async_copy(table_hbm.at[idx_v.at[0]], bufs[0], gsems[0])
    pltpu.async_copy(table_hbm.at[idx_v.at[1]], bufs[1], gsems[1])
    for c in range(_NCHUNK):
        r = c % 3
        out_slice = out_hbm.at[pl.ds(base + c * _K, _K)]
        pltpu.make_async_copy(table_hbm.at[idx_v.at[c]], bufs[r], gsems[r]).wait()
        pltpu.async_copy(bufs[r], out_slice, osems[r])
        if c + 2 < _NCHUNK:
            nr = (c + 2) % 3
            if c - 1 >= 0:
                prev_slice = out_hbm.at[pl.ds(base + (c - 1) * _K, _K)]
                pltpu.make_async_copy(bufs[nr], prev_slice, osems[nr]).wait()
            pltpu.async_copy(table_hbm.at[idx_v.at[c + 2]], bufs[nr], gsems[nr])
    for c in (_NCHUNK - 2, _NCHUNK - 1):
        r = c % 3
        out_slice = out_hbm.at[pl.ds(base + c * _K, _K)]
        pltpu.make_async_copy(bufs[r], out_slice, osems[r]).wait()


def kernel(idx, weight):
    batch, seq = idx.shape
    idx_grid = idx.reshape(_NW, _NCHUNK, _K)
    out = _gather_kernel(idx_grid, weight)
    return out.reshape(batch, seq, weight.shape[1])


# speedup vs baseline: 1.0023x; 1.0023x over previous
import functools

import jax
import jax.numpy as jnp
from jax import lax
from jax.experimental import pallas as pl
from jax.experimental.pallas import tpu as pltpu
from jax.experimental.pallas import tpu_sc as plsc

_VOCAB = 100000
_HIDDEN = 1024
_BATCH = 4
_SEQ = 2048
_B = _BATCH * _SEQ
_NC = 2
_NS = 16
_NW = _NC * _NS
_BPW = _B // _NW
_WPR = _SEQ // _BPW
_K = 32
_NCHUNK = _BPW // _K

_mesh = plsc.VectorSubcoreMesh(core_axis_name="c", subcore_axis_name="s")


@functools.partial(
    pl.kernel,
    mesh=_mesh,
    out_type=jax.ShapeDtypeStruct((_B, _HIDDEN), jnp.float32),
    scratch_types=[
        pltpu.VMEM((_BPW,), jnp.int32),
        pltpu.VMEM((_K, _HIDDEN), jnp.float32),
        pltpu.VMEM((_K, _HIDDEN), jnp.float32),
        pltpu.SemaphoreType.DMA,
        pltpu.SemaphoreType.DMA,
    ],
)
def _gather_kernel(idx_hbm, table_hbm, out_hbm, idx_v, buf0, buf1, sem0, sem1):
    wid = lax.axis_index("s") * _NC + lax.axis_index("c")
    base = wid * _BPW
    row = wid // _WPR
    col = (wid % _WPR) * _BPW
    pltpu.sync_copy(idx_hbm.at[row, pl.ds(col, _BPW)], idx_v)
    pltpu.async_copy(table_hbm.at[idx_v.at[pl.ds(0, _K)]], buf0, sem0)
    for c in range(_NCHUNK):
        buf, sem = (buf0, sem0) if c % 2 == 0 else (buf1, sem1)
        nbuf, nsem = (buf1, sem1) if c % 2 == 0 else (buf0, sem0)
        if c + 1 < _NCHUNK:
            pltpu.async_copy(
                table_hbm.at[idx_v.at[pl.ds((c + 1) * _K, _K)]], nbuf, nsem)
        pltpu.make_async_copy(
            table_hbm.at[idx_v.at[pl.ds(c * _K, _K)]], buf, sem).wait()
        pltpu.sync_copy(buf, out_hbm.at[pl.ds(base + c * _K, _K)])


def kernel(idx, weight):
    batch, seq = idx.shape
    out = _gather_kernel(idx, weight)
    return out.reshape(batch, seq, weight.shape[1])


# speedup vs baseline: 1.0112x; 1.0088x over previous
import functools

import jax
import jax.numpy as jnp
from jax import lax
from jax.experimental import pallas as pl
from jax.experimental.pallas import tpu as pltpu
from jax.experimental.pallas import tpu_sc as plsc

_VOCAB = 100000
_HIDDEN = 1024
_BATCH = 4
_SEQ = 2048
_B = _BATCH * _SEQ
_NC = 2
_NS = 16
_NW = _NC * _NS
_BPW = _B // _NW
_WPR = _SEQ // _BPW
_K = 16
_NCHUNK = _BPW // _K
_NBUF = 6
_DEPTH = 4

_mesh = plsc.VectorSubcoreMesh(core_axis_name="c", subcore_axis_name="s")


@functools.partial(
    pl.kernel,
    mesh=_mesh,
    out_type=jax.ShapeDtypeStruct((_B, _HIDDEN), jnp.float32),
    scratch_types=(
        [pltpu.VMEM((_BPW,), jnp.int32)]
        + [pltpu.VMEM((_K, _HIDDEN), jnp.float32) for _ in range(_NBUF)]
        + [pltpu.SemaphoreType.DMA for _ in range(2 * _NBUF)]
    ),
)
def _gather_kernel(idx_hbm, table_hbm, out_hbm, idx_v, *rest):
    bufs = rest[:_NBUF]
    gsems = rest[_NBUF:2 * _NBUF]
    osems = rest[2 * _NBUF:]
    wid = lax.axis_index("s") * _NC + lax.axis_index("c")
    base = wid * _BPW
    row = wid // _WPR
    col = (wid % _WPR) * _BPW
    pltpu.sync_copy(idx_hbm.at[row, pl.ds(col, _BPW)], idx_v)

    def gather(c):
        r = c % _NBUF
        pltpu.async_copy(
            table_hbm.at[idx_v.at[pl.ds(c * _K, _K)]], bufs[r], gsems[r])

    def out_slice(c):
        return out_hbm.at[pl.ds(base + c * _K, _K)]

    for c in range(_DEPTH):
        gather(c)
    for c in range(_NCHUNK):
        r = c % _NBUF
        pltpu.make_async_copy(
            table_hbm.at[idx_v.at[pl.ds(c * _K, _K)]], bufs[r], gsems[r]).wait()
        pltpu.async_copy(bufs[r], out_slice(c), osems[r])
        n = c + _DEPTH
        if n < _NCHUNK:
            nr = n % _NBUF
            prev = n - _NBUF
            if prev >= 0:
                pltpu.make_async_copy(bufs[nr], out_slice(prev), osems[nr]).wait()
            gather(n)
    for c in range(_NCHUNK - _NBUF, _NCHUNK):
        if c >= 0:
            r = c % _NBUF
            pltpu.make_async_copy(bufs[r], out_slice(c), osems[r]).wait()


def kernel(idx, weight):
    batch, seq = idx.shape
    out = _gather_kernel(idx, weight)
    return out.reshape(batch, seq, weight.shape[1])


# speedup vs baseline: 1.0436x; 1.0320x over previous
import functools

import jax
import jax.numpy as jnp
from jax import lax
from jax.experimental import pallas as pl
from jax.experimental.pallas import tpu as pltpu
from jax.experimental.pallas import tpu_sc as plsc

_VOCAB = 100000
_HIDDEN = 1024
_BATCH = 4
_SEQ = 2048
_B = _BATCH * _SEQ
_NC = 2
_NS = 16
_NW = _NC * _NS
_BPW = _B // _NW
_WPR = _SEQ // _BPW
_K = 16
_NCHUNK = _BPW // _K
_NBUF = 7
_DEPTH = 5

_mesh = plsc.VectorSubcoreMesh(core_axis_name="c", subcore_axis_name="s")


@functools.partial(
    pl.kernel,
    mesh=_mesh,
    out_type=jax.ShapeDtypeStruct((_B, _HIDDEN), jnp.float32),
    scratch_types=(
        [pltpu.VMEM((_BPW,), jnp.int32)]
        + [pltpu.VMEM((_K, _HIDDEN), jnp.float32) for _ in range(_NBUF)]
        + [pltpu.SemaphoreType.DMA for _ in range(2 * _NBUF)]
    ),
)
def _gather_kernel(idx_hbm, table_hbm, out_hbm, idx_v, *rest):
    bufs = rest[:_NBUF]
    gsems = rest[_NBUF:2 * _NBUF]
    osems = rest[2 * _NBUF:]
    wid = lax.axis_index("s") * _NC + lax.axis_index("c")
    base = wid * _BPW
    row = wid // _WPR
    col = (wid % _WPR) * _BPW
    pltpu.sync_copy(idx_hbm.at[row, pl.ds(col, _BPW)], idx_v)

    def gather(c):
        r = c % _NBUF
        pltpu.async_copy(
            table_hbm.at[idx_v.at[pl.ds(c * _K, _K)]], bufs[r], gsems[r])

    def out_slice(c):
        return out_hbm.at[pl.ds(base + c * _K, _K)]

    for c in range(_DEPTH):
        gather(c)
    for c in range(_NCHUNK):
        r = c % _NBUF
        pltpu.make_async_copy(
            table_hbm.at[idx_v.at[pl.ds(c * _K, _K)]], bufs[r], gsems[r]).wait()
        pltpu.async_copy(bufs[r], out_slice(c), osems[r])
        n = c + _DEPTH
        if n < _NCHUNK:
            nr = n % _NBUF
            prev = n - _NBUF
            if prev >= 0:
                pltpu.make_async_copy(bufs[nr], out_slice(prev), osems[nr]).wait()
            gather(n)
    for c in range(_NCHUNK - _NBUF, _NCHUNK):
        if c >= 0:
            r = c % _NBUF
            pltpu.make_async_copy(bufs[r], out_slice(c), osems[r]).wait()


def kernel(idx, weight):
    batch, seq = idx.shape
    out = _gather_kernel(idx, weight)
    return out.reshape(batch, seq, weight.shape[1])
